# baseline (speedup 1.0000x reference)
import functools

import jax
import jax.numpy as jnp
from jax import lax
from jax.experimental import pallas as pl
from jax.experimental.pallas import tpu as pltpu
from jax.experimental.pallas import tpu_sc as plsc

_INFO = plsc.get_sparse_core_info()
_NC, _NS = _INFO.num_cores, _INFO.num_subcores
_NW = _NC * _NS
_CHUNK = 1000


def _make_edges_copy(n_edges, d_edge, dtype):
    rows_per_w = n_edges // _NW
    n_chunks = rows_per_w // _CHUNK
    mesh = plsc.VectorSubcoreMesh(core_axis_name="c", subcore_axis_name="s")

    @functools.partial(
        pl.kernel,
        mesh=mesh,
        out_type=jax.ShapeDtypeStruct((n_edges, d_edge), dtype),
        scratch_types=[
            pltpu.VMEM((_CHUNK, d_edge), dtype),
            pltpu.SemaphoreType.DMA,
        ],
    )
    def k(e_hbm, out_hbm, buf, sem):
        wid = lax.axis_index("s") * _NC + lax.axis_index("c")
        base = wid * rows_per_w

        def body(c, carry):
            start = base + c * _CHUNK
            pltpu.sync_copy(e_hbm.at[pl.ds(start, _CHUNK)], buf)
            pltpu.sync_copy(buf, out_hbm.at[pl.ds(start, _CHUNK)])
            return carry

        lax.fori_loop(0, n_chunks, body, 0)

    return k


def kernel(nodes, edge_index, edges=None, u=None, batch=None):
    if batch is None:
        batch = jnp.zeros((nodes.shape[0],), dtype=jnp.int32)
    n_edges, d_edge = edges.shape
    edges_o = _make_edges_copy(n_edges, d_edge, edges.dtype)(edges)
    return (nodes, edge_index, edges_o, u, batch)
